# baseline (speedup 1.0000x reference)
import functools

import jax
import jax.numpy as jnp
from jax import lax
from jax.experimental import pallas as pl
from jax.experimental.pallas import tpu as pltpu
from jax.experimental.pallas import tpu_sc as plsc

M = 4
DEPTH = 6
LD = 256
B = 2048
NCODE = 64
TEMP = 0.07
TOTAL_ROWS = M * DEPTH * B
NMD = M * DEPTH


def _tc_route_body(lat_ref, e_ref, idx_ref, fidx_ref, oh_ref, en_ref,
                   vq_ref, dist_ref):
    f32 = jnp.float32
    inf = jnp.asarray(jnp.inf, f32)
    ones_ld = jnp.ones((1, LD), f32)

    vq_total = jnp.asarray(0.0, f32)
    for m in range(M):
        x = lat_ref[m]
        em = e_ref[m]
        xsq_t = lax.dot_general(ones_ld, x * x, (((1,), (1,)), ((), ())),
                                preferred_element_type=f32)
        esq = jnp.sum(em * em, axis=1, keepdims=True)
        g_t = lax.dot_general(em, x, (((1,), (1,)), ((), ())),
                              preferred_element_type=f32)
        dist_ref[...] = esq + xsq_t - 2.0 * g_t
        en_ref[m] = em * (1.0 / jnp.maximum(jnp.sqrt(esq), 1e-12))

        parent = jnp.zeros((1, B), jnp.int32)
        for d in range(DEPTH):
            o = 2 ** d
            k = 2 ** d
            dist = dist_ref[o:2 * o, :]
            rowi = lax.broadcasted_iota(jnp.int32, (k, B), 0)
            if d == 0:
                masked = dist
                minv = jnp.min(masked, axis=0, keepdims=True)
                eq = dist == minv
            else:
                allowed = (rowi >> 1) == parent
                masked = jnp.where(allowed, dist, inf)
                minv = jnp.min(masked, axis=0, keepdims=True)
                eq = allowed & (dist == minv)
            argloc = jnp.min(jnp.where(eq, rowi, NCODE),
                             axis=0, keepdims=True)
            r = m * DEPTH + d
            idx_ref[r, :] = jnp.reshape(argloc, (B,))
            fidx_ref[r, :] = jnp.reshape(argloc + (m * 64 + o - 1), (B,))
            oh_ref[m, o:2 * o, :] = (rowi == argloc).astype(f32)

            ec_sum = jnp.sum(minv)
            colmin = jnp.min(masked, axis=1, keepdims=True)
            if d == 0:
                ce_vals = colmin
            else:
                present = jnp.any(allowed, axis=1, keepdims=True)
                ce_vals = jnp.where(present, colmin, dist[:, 0:1])
            ce_sum = jnp.sum(ce_vals)
            vq_total = (vq_total + 2.0 * ec_sum / (B * LD)
                        + 2.0 * ce_sum / (k * LD))
            parent = argloc

        oh_ref[m, 0:1, :] = jnp.zeros((1, B), f32)

    vq_ref[...] = jnp.reshape(vq_total, (1, 1))


def _tc_loss_body(oh_ref, en_ref, tab_ref, vq_ref, loss_ref):
    f32 = jnp.float32
    col1 = lax.broadcasted_iota(jnp.int32, (1, NCODE), 1)
    rcol = lax.broadcasted_iota(jnp.int32, (NCODE, 1), 0)
    ii2 = lax.broadcasted_iota(jnp.int32, (NCODE, NCODE), 0)
    jj2 = lax.broadcasted_iota(jnp.int32, (NCODE, NCODE), 1)
    inf = jnp.asarray(jnp.inf, f32)
    ones_row = jnp.ones((1, B), f32)

    cnt_row = []
    cnt_col = []
    for m in range(M):
        cnt_row.append(lax.dot_general(
            ones_row, oh_ref[m], (((1,), (1,)), ((), ())),
            preferred_element_type=f32))
        cnt_col.append(lax.dot_general(
            oh_ref[m], jnp.ones((B, 1), f32), (((1,), (0,)), ((), ())),
            preferred_element_type=f32))
    vq_total = vq_ref[...]

    total_kl = jnp.asarray(0.0, f32)
    rows32 = lax.broadcasted_iota(jnp.int32, (32, LD), 0)
    for m in range(M):
        for d in range(DEPTH):
            k = 2 ** d
            mu = tab_ref[m, d, :, :LD]
            logv = tab_ref[m, d, :, LD:]
            term = 0.5 * (jnp.exp(logv) + mu * mu - 1.0 - logv)
            total_kl = total_kl + jnp.sum(
                jnp.where(rows32 < k, term, 0.0))
    kl_loss = total_kl / ((63 * M) * LD)

    s_mats, j_mats = [], []
    for m in range(M - 1):
        s_mats.append(lax.dot_general(
            en_ref[m], en_ref[m + 1], (((1,), (1,)), ((), ())),
            preferred_element_type=f32))
        j_mats.append(lax.dot_general(
            oh_ref[m], oh_ref[m + 1], (((1,), (1,)), ((), ())),
            preferred_element_type=f32))

    align = jnp.asarray(0.0, f32)
    inv_b = 1.0 / B
    for d in range(DEPTH):
        o = 2 ** d
        k = 2 ** d
        lvl_r = (col1 >= o) & (col1 < 2 * o)
        lvl_c = (rcol >= o) & (rcol < 2 * o)
        lvl2 = ((ii2 >= o) & (ii2 < 2 * o)
                & (jj2 >= o) & (jj2 < 2 * o))
        for m in range(M - 1):
            cm_r, cm_c = cnt_row[m], cnt_col[m]
            cn_r = cnt_row[m + 1]
            if d >= 1:
                e_r = (cm_r - cn_r) * (1.0 / B)
                ebc = jnp.broadcast_to(e_r, (NCODE, NCODE))
                ot = jnp.asarray(0.0, f32)
                for s in range(d):
                    agg = ((jj2 >= o) & (jj2 < 2 * o)
                           & (ii2 == ((jj2 - o) >> s)))
                    w = jnp.sum(jnp.where(agg, ebc, 0.0),
                                axis=1, keepdims=True)
                    ot = ot + jnp.sum(jnp.abs(w))
                align = align + ot
            st = s_mats[m] * (1.0 / TEMP)
            colok = lvl_r & (cn_r > 0)
            sm = jnp.where(colok, st, -inf)
            mx = jnp.max(sm, axis=1, keepdims=True)
            ex = jnp.where(colok, cn_r * jnp.exp(sm - mx), 0.0)
            lse = mx + jnp.log(jnp.sum(ex, axis=1, keepdims=True))
            mean_lse = jnp.sum(
                jnp.where(lvl_c, cm_c * lse, 0.0)) * inv_b
            mean_diag = jnp.sum(
                jnp.where(lvl2, j_mats[m] * st, 0.0)) * inv_b
            align = align + (mean_lse - mean_diag)

    loss_ref[...] = jnp.reshape(vq_total + kl_loss + align, (1, 1))


def _tc_route(latents, e_all):
    return pl.pallas_call(
        _tc_route_body,
        out_shape=(
            jax.ShapeDtypeStruct((NMD, B), jnp.int32),
            jax.ShapeDtypeStruct((NMD, B), jnp.int32),
            jax.ShapeDtypeStruct((M, NCODE, B), jnp.float32),
            jax.ShapeDtypeStruct((M, NCODE, LD), jnp.float32),
            jax.ShapeDtypeStruct((1, 1), jnp.float32),
        ),
        scratch_shapes=[
            pltpu.VMEM((NCODE, B), jnp.float32),
        ],
    )(latents, e_all)


def _tc_loss(oh, en, tables, vq):
    return pl.pallas_call(
        _tc_loss_body,
        out_shape=jax.ShapeDtypeStruct((1, 1), jnp.float32),
    )(oh, en, tables, vq)


_NW = 32
_RPW = TOTAL_ROWS // _NW
_CH = 96
_NCH = _RPW // _CH
_NTAB = 256


def _sc_zgather(codes_compact, fidx3d):
    mesh = plsc.VectorSubcoreMesh(core_axis_name="c", subcore_axis_name="s")

    @functools.partial(
        pl.kernel, mesh=mesh,
        out_type=jax.ShapeDtypeStruct((TOTAL_ROWS, LD), jnp.float32),
        scratch_types=[
            pltpu.VMEM((_NTAB, LD), jnp.float32),
            pltpu.VMEM((_NCH, _CH), jnp.int32),
            pltpu.VMEM((_CH, LD), jnp.float32),
            pltpu.VMEM((_CH, LD), jnp.float32),
            pltpu.SemaphoreType.DMA,
            pltpu.SemaphoreType.DMA,
            pltpu.SemaphoreType.DMA,
        ],
    )
    def body(codes_hbm, fidx_hbm, out_hbm, tab_v, idx_v, buf0, buf1,
             sem0, sem1, semg):
        c = lax.axis_index("c")
        s = lax.axis_index("s")
        wid = s * 2 + c
        pltpu.sync_copy(fidx_hbm.at[wid], idx_v)
        pltpu.sync_copy(codes_hbm, tab_v)
        bufs = (buf0, buf1)
        sems = (sem0, sem1)

        def fill(j, buf):
            @plsc.parallel_loop(0, _CH // 16)
            def _group(g):
                vec = idx_v[j, pl.ds(pl.multiple_of(g * 16, 16), 16)]
                for l in range(0, 16, 2):
                    row0 = vec[l]
                    row1 = vec[l + 1]
                    vals0 = [tab_v[row0, pl.ds(cc * 16, 16)]
                             for cc in range(LD // 16)]
                    vals1 = [tab_v[row1, pl.ds(cc * 16, 16)]
                             for cc in range(LD // 16)]
                    for cc in range(LD // 16):
                        buf[g * 16 + l, pl.ds(cc * 16, 16)] = vals0[cc]
                    for cc in range(LD // 16):
                        buf[g * 16 + l + 1, pl.ds(cc * 16, 16)] = vals1[cc]

        def flush(j, buf, sem):
            return pltpu.async_copy(
                buf, out_hbm.at[pl.ds(wid * _RPW + j * _CH, _CH)], sem)

        fill(0, buf0)
        p0 = flush(0, buf0, sem0)
        fill(1, buf1)
        p1 = flush(1, buf1, sem1)

        def pair_body(p, _):
            j0 = 2 * p
            p0.wait()
            fill(j0, buf0)
            flush(j0, buf0, sem0)
            p1.wait()
            fill(j0 + 1, buf1)
            flush(j0 + 1, buf1, sem1)
            return 0

        lax.fori_loop(1, _NCH // 2, pair_body, 0)
        p0.wait()
        p1.wait()

    return body(codes_compact, fidx3d)


def kernel(latents_in, tree_tables):
    parts = [jnp.zeros((M, 1, LD), jnp.float32)]
    for d in range(DEPTH):
        parts.append(tree_tables[:, d, :2 ** d, :LD])
    e_all = jnp.concatenate(parts, axis=1)

    idx24, fidx24, oh, en, vq = _tc_route(latents_in, e_all)
    loss = _tc_loss(oh, en, tree_tables, vq)

    cparts = [tree_tables[:, d, :2 ** d, :LD] for d in range(DEPTH)]
    cparts.append(jnp.zeros((M, 1, LD), jnp.float32))
    codes_compact = jnp.concatenate(cparts, axis=1).reshape(_NTAB, LD)
    z_flat = _sc_zgather(codes_compact, fidx24.reshape(_NW, _NCH, _CH))

    idx_out = idx24.reshape(M, DEPTH, B)
    z_out = z_flat.reshape(M, DEPTH, B, LD)
    return idx_out, z_out, loss.reshape(())


# speedup vs baseline: 1.0056x; 1.0056x over previous
import functools

import jax
import jax.numpy as jnp
from jax import lax
from jax.experimental import pallas as pl
from jax.experimental.pallas import tpu as pltpu
from jax.experimental.pallas import tpu_sc as plsc

M = 4
DEPTH = 6
LD = 256
B = 2048
NCODE = 64
TEMP = 0.07
TOTAL_ROWS = M * DEPTH * B
NMD = M * DEPTH


def _tc_route_body(lat_ref, e_ref, idx_ref, fidx_ref, oh_ref, en_ref,
                   vq_ref, dist_ref):
    f32 = jnp.float32
    inf = jnp.asarray(jnp.inf, f32)
    ones_ld = jnp.ones((1, LD), f32)

    vq_total = jnp.asarray(0.0, f32)
    for m in range(M):
        x = lat_ref[m]
        em = e_ref[m]
        xsq_t = lax.dot_general(ones_ld, x * x, (((1,), (1,)), ((), ())),
                                preferred_element_type=f32)
        esq = jnp.sum(em * em, axis=1, keepdims=True)
        g_t = lax.dot_general(em, x, (((1,), (1,)), ((), ())),
                              preferred_element_type=f32)
        dist_ref[...] = esq + xsq_t - 2.0 * g_t
        en_ref[m] = em * (1.0 / jnp.maximum(jnp.sqrt(esq), 1e-12))

        parent = jnp.zeros((1, B), jnp.int32)
        for d in range(DEPTH):
            o = 2 ** d
            k = 2 ** d
            dist = dist_ref[o:2 * o, :]
            rowi = lax.broadcasted_iota(jnp.int32, (k, B), 0)
            if d == 0:
                masked = dist
                minv = jnp.min(masked, axis=0, keepdims=True)
                eq = dist == minv
            else:
                allowed = (rowi >> 1) == parent
                masked = jnp.where(allowed, dist, inf)
                minv = jnp.min(masked, axis=0, keepdims=True)
                eq = allowed & (dist == minv)
            argloc = jnp.min(jnp.where(eq, rowi, NCODE),
                             axis=0, keepdims=True)
            r = m * DEPTH + d
            idx_ref[r, :] = jnp.reshape(argloc, (B,))
            fidx_ref[r, :] = jnp.reshape(argloc + (m * 64 + o - 1), (B,))
            oh_ref[m, o:2 * o, :] = (rowi == argloc).astype(f32)

            ec_sum = jnp.sum(minv)
            colmin = jnp.min(masked, axis=1, keepdims=True)
            if d == 0:
                ce_vals = colmin
            else:
                present = jnp.any(allowed, axis=1, keepdims=True)
                ce_vals = jnp.where(present, colmin, dist[:, 0:1])
            ce_sum = jnp.sum(ce_vals)
            vq_total = (vq_total + 2.0 * ec_sum / (B * LD)
                        + 2.0 * ce_sum / (k * LD))
            parent = argloc

        oh_ref[m, 0:1, :] = jnp.zeros((1, B), f32)

    vq_ref[...] = jnp.reshape(vq_total, (1, 1))


def _tc_loss_body(oh_ref, en_ref, tab_ref, vq_ref, loss_ref):
    f32 = jnp.float32
    col1 = lax.broadcasted_iota(jnp.int32, (1, NCODE), 1)
    rcol = lax.broadcasted_iota(jnp.int32, (NCODE, 1), 0)
    ii2 = lax.broadcasted_iota(jnp.int32, (NCODE, NCODE), 0)
    jj2 = lax.broadcasted_iota(jnp.int32, (NCODE, NCODE), 1)
    inf = jnp.asarray(jnp.inf, f32)
    ones_row = jnp.ones((1, B), f32)

    cnt_row = []
    cnt_col = []
    for m in range(M):
        cnt_row.append(lax.dot_general(
            ones_row, oh_ref[m], (((1,), (1,)), ((), ())),
            preferred_element_type=f32))
        cnt_col.append(lax.dot_general(
            oh_ref[m], jnp.ones((B, 1), f32), (((1,), (0,)), ((), ())),
            preferred_element_type=f32))
    vq_total = vq_ref[...]

    total_kl = jnp.asarray(0.0, f32)
    rows32 = lax.broadcasted_iota(jnp.int32, (32, LD), 0)
    for m in range(M):
        for d in range(DEPTH):
            k = 2 ** d
            mu = tab_ref[m, d, :, :LD]
            logv = tab_ref[m, d, :, LD:]
            term = 0.5 * (jnp.exp(logv) + mu * mu - 1.0 - logv)
            total_kl = total_kl + jnp.sum(
                jnp.where(rows32 < k, term, 0.0))
    kl_loss = total_kl / ((63 * M) * LD)

    s_mats, j_mats = [], []
    for m in range(M - 1):
        s_mats.append(lax.dot_general(
            en_ref[m], en_ref[m + 1], (((1,), (1,)), ((), ())),
            preferred_element_type=f32))
        j_mats.append(lax.dot_general(
            oh_ref[m], oh_ref[m + 1], (((1,), (1,)), ((), ())),
            preferred_element_type=f32))

    align = jnp.asarray(0.0, f32)
    inv_b = 1.0 / B
    for d in range(DEPTH):
        o = 2 ** d
        k = 2 ** d
        lvl_r = (col1 >= o) & (col1 < 2 * o)
        lvl_c = (rcol >= o) & (rcol < 2 * o)
        lvl2 = ((ii2 >= o) & (ii2 < 2 * o)
                & (jj2 >= o) & (jj2 < 2 * o))
        for m in range(M - 1):
            cm_r, cm_c = cnt_row[m], cnt_col[m]
            cn_r = cnt_row[m + 1]
            if d >= 1:
                e_r = (cm_r - cn_r) * (1.0 / B)
                ebc = jnp.broadcast_to(e_r, (NCODE, NCODE))
                ot = jnp.asarray(0.0, f32)
                for s in range(d):
                    agg = ((jj2 >= o) & (jj2 < 2 * o)
                           & (ii2 == ((jj2 - o) >> s)))
                    w = jnp.sum(jnp.where(agg, ebc, 0.0),
                                axis=1, keepdims=True)
                    ot = ot + jnp.sum(jnp.abs(w))
                align = align + ot
            st = s_mats[m] * (1.0 / TEMP)
            colok = lvl_r & (cn_r > 0)
            sm = jnp.where(colok, st, -inf)
            mx = jnp.max(sm, axis=1, keepdims=True)
            ex = jnp.where(colok, cn_r * jnp.exp(sm - mx), 0.0)
            lse = mx + jnp.log(jnp.sum(ex, axis=1, keepdims=True))
            mean_lse = jnp.sum(
                jnp.where(lvl_c, cm_c * lse, 0.0)) * inv_b
            mean_diag = jnp.sum(
                jnp.where(lvl2, j_mats[m] * st, 0.0)) * inv_b
            align = align + (mean_lse - mean_diag)

    loss_ref[...] = jnp.reshape(vq_total + kl_loss + align, (1, 1))


def _tc_route(latents, e_all):
    return pl.pallas_call(
        _tc_route_body,
        out_shape=(
            jax.ShapeDtypeStruct((NMD, B), jnp.int32),
            jax.ShapeDtypeStruct((NMD, B), jnp.int32),
            jax.ShapeDtypeStruct((M, NCODE, B), jnp.float32),
            jax.ShapeDtypeStruct((M, NCODE, LD), jnp.float32),
            jax.ShapeDtypeStruct((1, 1), jnp.float32),
        ),
        scratch_shapes=[
            pltpu.VMEM((NCODE, B), jnp.float32),
        ],
    )(latents, e_all)


def _tc_loss(oh, en, tables, vq):
    return pl.pallas_call(
        _tc_loss_body,
        out_shape=jax.ShapeDtypeStruct((1, 1), jnp.float32),
    )(oh, en, tables, vq)


_NW = 32
_RPW = TOTAL_ROWS // _NW
_CH = 96
_NCH = _RPW // _CH
_NTAB = 256


def _sc_zgather(codes_compact, fidx3d):
    mesh = plsc.VectorSubcoreMesh(core_axis_name="c", subcore_axis_name="s")

    @functools.partial(
        pl.kernel, mesh=mesh,
        out_type=jax.ShapeDtypeStruct((TOTAL_ROWS, LD), jnp.float32),
        scratch_types=[
            pltpu.VMEM((_NTAB, LD), jnp.float32),
            pltpu.VMEM((_NCH, _CH), jnp.int32),
            pltpu.VMEM((_CH, LD), jnp.float32),
            pltpu.VMEM((_CH, LD), jnp.float32),
            pltpu.SemaphoreType.DMA,
            pltpu.SemaphoreType.DMA,
            pltpu.SemaphoreType.DMA,
        ],
    )
    def body(codes_hbm, fidx_hbm, out_hbm, tab_v, idx_v, buf0, buf1,
             sem0, sem1, semg):
        c = lax.axis_index("c")
        s = lax.axis_index("s")
        wid = s * 2 + c
        pltpu.sync_copy(fidx_hbm.at[wid], idx_v)
        pltpu.sync_copy(codes_hbm, tab_v)
        bufs = (buf0, buf1)
        sems = (sem0, sem1)

        def fill(j, buf):
            @plsc.parallel_loop(0, _CH // 16)
            def _group(g):
                vec = idx_v[j, pl.ds(pl.multiple_of(g * 16, 16), 16)]
                for l in range(16):
                    row = vec[l]
                    vals = [tab_v[row, pl.ds(cc * 16, 16)]
                            for cc in range(LD // 16)]
                    for cc in range(LD // 16):
                        buf[g * 16 + l, pl.ds(cc * 16, 16)] = vals[cc]

        def flush(j, buf, sem):
            return pltpu.async_copy(
                buf, out_hbm.at[pl.ds(wid * _RPW + j * _CH, _CH)], sem)

        fill(0, buf0)
        p0 = flush(0, buf0, sem0)
        fill(1, buf1)
        p1 = flush(1, buf1, sem1)

        def pair_body(p, _):
            j0 = 2 * p
            p0.wait()
            fill(j0, buf0)
            flush(j0, buf0, sem0)
            p1.wait()
            fill(j0 + 1, buf1)
            flush(j0 + 1, buf1, sem1)
            return 0

        lax.fori_loop(1, _NCH // 2, pair_body, 0)
        p0.wait()
        p1.wait()

    return body(codes_compact, fidx3d)


def kernel(latents_in, tree_tables):
    parts = [jnp.zeros((M, 1, LD), jnp.float32)]
    for d in range(DEPTH):
        parts.append(tree_tables[:, d, :2 ** d, :LD])
    e_all = jnp.concatenate(parts, axis=1)

    idx24, fidx24, oh, en, vq = _tc_route(latents_in, e_all)
    loss = _tc_loss(oh, en, tree_tables, vq)

    cparts = [tree_tables[:, d, :2 ** d, :LD] for d in range(DEPTH)]
    cparts.append(jnp.zeros((M, 1, LD), jnp.float32))
    codes_compact = jnp.concatenate(cparts, axis=1).reshape(_NTAB, LD)
    z_flat = _sc_zgather(codes_compact, fidx24.reshape(_NW, _NCH, _CH))

    idx_out = idx24.reshape(M, DEPTH, B)
    z_out = z_flat.reshape(M, DEPTH, B, LD)
    return idx_out, z_out, loss.reshape(())
